# Initial kernel scaffold; baseline (speedup 1.0000x reference)
import jax
import jax.numpy as jnp
from jax.experimental import pallas as pl


def kernel(x, edge_index, edge_attr, fcn_W, fcn_b, fce_W, fce_b, bn_g, bn_b, fc_W0, fc_b0, attn_W0, attn_b0, fc_W1, fc_b1, attn_W1, attn_b1, fc_W2, fc_b2, attn_W2, attn_b2):
    raise NotImplementedError("write your pallas kernel here")


# speedup vs baseline: 7.2550x; 7.2550x over previous
import functools

import jax
import jax.numpy as jnp
from jax import lax
from jax.experimental import pallas as pl
from jax.experimental.pallas import tpu as pltpu
from jax.experimental.pallas import tpu_sc as plsc

_N = 10000
_E = 320000
_D = 128
_DE = 16
_H = _D // 2

_NC = 2
_NS = 16
_B = 32
_EPT = _E // _NS
_ZR = 632
_LASTR = _N - (_NS - 1) * _ZR

_BN_BLK = 1000
_BE_BLK = 4000


def _fold_body(x_ref, fcnW_ref, fcnb_ref, bng_ref, bnb_ref,
               fceW_ref, fceb_ref, fcW_ref, fcb_ref,
               Ai_ref, Aj_ref, Ae_ref, ab_ref,
               Wh_ref, bh_ref, Wi_ref, bi_ref, Wj_ref, bj_ref,
               M_ref, cq_ref):
    n = jnp.float32(_N)
    xb = x_ref[...]
    S = lax.dot_general(xb, xb, (((0,), (0,)), ((), ())),
                        preferred_element_type=jnp.float32)
    mx = jnp.sum(xb, axis=0, keepdims=True) / n
    C = S / n - mx.T @ mx
    fcnW = fcnW_ref[...]
    mh = mx @ fcnW.T + fcnb_ref[...]
    t1 = fcnW @ C
    var = jnp.sum(t1 * fcnW, axis=1, keepdims=True).T
    scale = bng_ref[...] / jnp.sqrt(var + 1e-5)
    shift = bnb_ref[...] - mh * scale
    W2 = fcnW.T * scale
    b2 = fcnb_ref[...] * scale + shift
    fceW = fceW_ref[...]
    fceb = fceb_ref[...]
    for l in range(3):
        fcW = fcW_ref[l]
        fcb = fcb_ref[l]
        if l == 0:
            Wh = W2 @ fcW.T
            bh = b2 @ fcW.T + fcb
        else:
            Wh = fcW.T
            bh = fcb
        Ai = Ai_ref[l]
        Aj = Aj_ref[l]
        Ae = Ae_ref[l]
        Wh_ref[l] = Wh
        bh_ref[l] = bh
        Wi_ref[l] = Wh @ Ai.T
        bi_ref[l] = bh @ Ai.T
        Wj_ref[l] = Wh @ Aj.T
        bj_ref[l] = bh @ Aj.T
        M_ref[l] = (Ae @ fceW).T
        cq_ref[l] = fceb @ Ae.T + ab_ref[l]


def _fold(x, fcnW, fcnb, bng, bnb, fceW, fceb, fcW, fcb, Ai, Aj, Ae, ab):
    full = lambda shp: pl.BlockSpec(shp, lambda: tuple(0 for _ in shp))
    ins = [x, fcnW, fcnb, bng, bnb, fceW, fceb, fcW, fcb, Ai, Aj, Ae, ab]
    return pl.pallas_call(
        _fold_body,
        in_specs=[full(a.shape) for a in ins],
        out_specs=[full((3, _D, _D)), full((3, 1, _D)),
                   full((3, _D, _D)), full((3, 1, _D)),
                   full((3, _D, _D)), full((3, 1, _D)),
                   full((3, _DE, _D)), full((3, 1, _D))],
        out_shape=[jax.ShapeDtypeStruct((3, _D, _D), jnp.float32),
                   jax.ShapeDtypeStruct((3, 1, _D), jnp.float32),
                   jax.ShapeDtypeStruct((3, _D, _D), jnp.float32),
                   jax.ShapeDtypeStruct((3, 1, _D), jnp.float32),
                   jax.ShapeDtypeStruct((3, _D, _D), jnp.float32),
                   jax.ShapeDtypeStruct((3, 1, _D), jnp.float32),
                   jax.ShapeDtypeStruct((3, _DE, _D), jnp.float32),
                   jax.ShapeDtypeStruct((3, 1, _D), jnp.float32)],
        interpret=False,
    )(*ins)


def _q_body(ea_ref, M_ref, cq_ref, q0_ref, q1_ref, q2_ref):
    ea = ea_ref[...]
    for l, qr in enumerate((q0_ref, q1_ref, q2_ref)):
        q = ea @ M_ref[l] + cq_ref[l]
        qr[0] = q[:, :_H]
        qr[1] = q[:, _H:]


def _qkern(ea, M, cq):
    qspec = pl.BlockSpec((2, _BE_BLK, _H), lambda i: (0, i, 0))
    qshape = jax.ShapeDtypeStruct((2, _E, _H), jnp.float32)
    return pl.pallas_call(
        _q_body,
        grid=(_E // _BE_BLK,),
        in_specs=[pl.BlockSpec((_BE_BLK, _DE), lambda i: (i, 0)),
                  pl.BlockSpec((3, _DE, _D), lambda i: (0, 0, 0)),
                  pl.BlockSpec((3, 1, _D), lambda i: (0, 0, 0))],
        out_specs=[qspec, qspec, qspec],
        out_shape=[qshape, qshape, qshape],
        interpret=False,
    )(ea, M, cq)


def _tables0_body(x_ref, Wh_ref, bh_ref, Wi_ref, bi_ref, Wj_ref, bj_ref,
                  stbl_ref, dtbl_ref):
    x = x_ref[...]
    h = x @ Wh_ref[...] + bh_ref[...]
    piv = x @ Wi_ref[...] + bi_ref[...]
    pjv = x @ Wj_ref[...] + bj_ref[...]
    stbl_ref[0] = jnp.concatenate([h[:, :_H], pjv[:, :_H]], axis=1)
    stbl_ref[1] = jnp.concatenate([h[:, _H:], pjv[:, _H:]], axis=1)
    dtbl_ref[...] = piv


def _tablesL_body(acc_ref, Wh_ref, bh_ref, Wi_ref, bi_ref, Wj_ref, bj_ref,
                  stbl_ref, dtbl_ref):
    a0 = acc_ref[0]
    a1 = acc_ref[1]
    x = jnp.concatenate([a0[:, :_H] / (a0[:, _H:] + 1e-16),
                         a1[:, :_H] / (a1[:, _H:] + 1e-16)], axis=1)
    h = x @ Wh_ref[...] + bh_ref[...]
    piv = x @ Wi_ref[...] + bi_ref[...]
    pjv = x @ Wj_ref[...] + bj_ref[...]
    stbl_ref[0] = jnp.concatenate([h[:, :_H], pjv[:, :_H]], axis=1)
    stbl_ref[1] = jnp.concatenate([h[:, _H:], pjv[:, _H:]], axis=1)
    dtbl_ref[...] = piv


def _tables(h_or_acc, Wh, bh, Wi, bi, Wj, bj, first):
    body = _tables0_body if first else _tablesL_body
    in0 = (pl.BlockSpec((_BN_BLK, _D), lambda i: (i, 0)) if first
           else pl.BlockSpec((2, _BN_BLK, _D), lambda i: (0, i, 0)))
    wspec = pl.BlockSpec((_D, _D), lambda i: (0, 0))
    bspec = pl.BlockSpec((1, _D), lambda i: (0, 0))
    return pl.pallas_call(
        body,
        grid=(_N // _BN_BLK,),
        in_specs=[in0, wspec, bspec, wspec, bspec, wspec, bspec],
        out_specs=[pl.BlockSpec((2, _BN_BLK, _D), lambda i: (0, i, 0)),
                   pl.BlockSpec((_BN_BLK, _D), lambda i: (i, 0))],
        out_shape=[jax.ShapeDtypeStruct((2, _N, _D), jnp.float32),
                   jax.ShapeDtypeStruct((_N, _D), jnp.float32)],
        interpret=False,
    )(h_or_acc, Wh, bh, Wi, bi, Wj, bj)


def _final_body(acc_ref, out_ref):
    a0 = acc_ref[0]
    a1 = acc_ref[1]
    out_ref[...] = jnp.concatenate([a0[:, :_H] / (a0[:, _H:] + 1e-16),
                                    a1[:, :_H] / (a1[:, _H:] + 1e-16)], axis=1)


def _final(acc):
    return pl.pallas_call(
        _final_body,
        grid=(_N // _BN_BLK,),
        in_specs=[pl.BlockSpec((2, _BN_BLK, _D), lambda i: (0, i, 0))],
        out_specs=pl.BlockSpec((_BN_BLK, _D), lambda i: (i, 0)),
        out_shape=jax.ShapeDtypeStruct((_N, _D), jnp.float32),
        interpret=False,
    )(acc)


def _sc_pass_body(src_ref, dst_ref, stbl_ref, dtbl_ref, q_ref, out_ref,
                  si0, si1, di0, di1, db0, db1, r0, r1, p0, p1, qb0, qb1,
                  o0, o1, acc,
                  sa0, sa1, sd0, sd1, sr0, sr1, sp0, sp1, sq0, sq1, ss0, ss1):
    c = lax.axis_index("c")
    s = lax.axis_index("s")
    coff = c * _N
    choff = c * _H
    nch = _EPT // _B
    ebase = s * _EPT
    qbase = c * _E + ebase

    si_b = (si0, si1)
    di_b = (di0, di1)
    db_b = (db0, db1)
    rows_b = (r0, r1)
    pd_b = (p0, p1)
    qv_b = (qb0, qb1)
    ov_b = (o0, o1)
    sem_si = (sa0, sa1)
    sem_di = (sd0, sd1)
    sem_r = (sr0, sr1)
    sem_p = (sp0, sp1)
    sem_q = (sq0, sq1)
    sem_s = (ss0, ss1)

    def zf(i, _):
        for k in range(_D // 16):
            o0[i, pl.ds(16 * k, 16)] = jnp.zeros((16,), jnp.float32)
        return 0
    lax.fori_loop(0, _B, zf, 0)

    nz8 = jnp.where(s == _NS - 1, _LASTR // 8, _ZR // 8)

    def zc(i, _):
        pltpu.async_copy(o0.at[pl.ds(0, 8)],
                         acc.at[pl.ds(s * _ZR + 8 * i, 8)], sa0)
        return 0
    lax.fori_loop(0, nz8, zc, 0)

    def zw(i, _):
        pltpu.make_async_copy(o0.at[pl.ds(0, 8)],
                              acc.at[pl.ds(s * _ZR, 8)], sa0).wait()
        return 0
    lax.fori_loop(0, nz8, zw, 0)
    plsc.subcore_barrier()

    def issue_idx(j, t):
        off = ebase + j * _B
        pltpu.async_copy(src_ref.at[pl.ds(off, _B)], si_b[t], sem_si[t])
        pltpu.async_copy(dst_ref.at[pl.ds(off, _B)], di_b[t], sem_di[t])

    def wait_idx_and_adjust(t):
        pltpu.make_async_copy(src_ref.at[pl.ds(0, _B)], si_b[t],
                              sem_si[t]).wait()
        pltpu.make_async_copy(dst_ref.at[pl.ds(0, _B)], di_b[t],
                              sem_di[t]).wait()
        for k in range(_B // 16):
            sl = pl.ds(16 * k, 16)
            si_b[t][sl] = si_b[t][sl] + coff

    def issue_gathers(j, t):
        pltpu.async_copy(stbl_ref.at[si_b[t]], rows_b[t], sem_r[t])
        pltpu.async_copy(dtbl_ref.at[di_b[t]], pd_b[t], sem_p[t])
        pltpu.async_copy(q_ref.at[pl.ds(qbase + j * _B, _B)], qv_b[t],
                         sem_q[t])

    def wait_gathers(t):
        pltpu.make_async_copy(stbl_ref.at[si_b[t]], rows_b[t], sem_r[t]).wait()
        pltpu.make_async_copy(dtbl_ref.at[di_b[t]], pd_b[t], sem_p[t]).wait()
        pltpu.make_async_copy(q_ref.at[pl.ds(qbase, _B)], qv_b[t],
                              sem_q[t]).wait()

    def compute(t):
        rr = rows_b[t]
        pp = pd_b[t]
        qq = qv_b[t]
        oo = ov_b[t]

        @plsc.parallel_loop(0, _B, 1, unroll=2)
        def _(i2):
            for k in range(_H // 16):
                lo = pl.ds(16 * k, 16)
                hi = pl.ds(_H + 16 * k, 16)
                a = pp[i2, pl.ds(choff + 16 * k, 16)] + rr[i2, hi] + qq[i2, lo]
                a = jnp.maximum(a, a * 0.2)
                ex = jnp.exp(a)
                oo[i2, lo] = rr[i2, lo] * ex
                oo[i2, hi] = ex

    def wait_scatter(t):
        pltpu.make_async_copy(ov_b[t], acc.at[db_b[t]], sem_s[t]).wait()

    def copy_db(t):
        for k in range(_B // 16):
            sl = pl.ds(16 * k, 16)
            db_b[t][sl] = di_b[t][sl]

    issue_idx(0, 0)
    wait_idx_and_adjust(0)
    issue_gathers(0, 0)
    issue_idx(1, 1)

    def body(i, _):
        for b in (0, 1):
            j = 2 * i + b
            t = b
            nt = 1 - b
            wait_idx_and_adjust(nt)
            issue_gathers(j + 1, nt)

            @pl.when(i > 0)
            def _():
                wait_scatter(t)
            wait_gathers(t)
            copy_db(t)
            if b == 0:
                issue_idx(j + 2, t)
            else:
                @pl.when(i < (nch - 1) // 2 - 1)
                def _():
                    issue_idx(j + 2, t)
            compute(t)
            pltpu.async_copy(ov_b[t], acc.at[db_b[t]], sem_s[t], add=True)
        return 0
    lax.fori_loop(0, (nch - 1) // 2, body, 0)

    wait_scatter(0)
    wait_gathers(0)
    copy_db(0)
    compute(0)
    pltpu.async_copy(ov_b[0], acc.at[db_b[0]], sem_s[0], add=True)
    wait_scatter(1)
    wait_scatter(0)
    plsc.subcore_barrier()

    @pl.when(s < _NS - 1)
    def _():
        pltpu.sync_copy(acc.at[pl.ds(s * _ZR, _ZR)],
                        out_ref.at[pl.ds(coff + s * _ZR, _ZR)])

    @pl.when(s == _NS - 1)
    def _():
        pltpu.sync_copy(acc.at[pl.ds((_NS - 1) * _ZR, _LASTR)],
                        out_ref.at[pl.ds(coff + (_NS - 1) * _ZR, _LASTR)])


def _sc_pass(src, dst, stbl, dtbl, q):
    mesh = plsc.VectorSubcoreMesh(core_axis_name="c", subcore_axis_name="s",
                                  num_cores=_NC, num_subcores=_NS)
    k = pl.kernel(
        _sc_pass_body,
        out_type=jax.ShapeDtypeStruct((2 * _N, _D), jnp.float32),
        mesh=mesh,
        scratch_types=(
            [pltpu.VMEM((_B,), jnp.int32)] * 6
            + [pltpu.VMEM((_B, _D), jnp.float32)] * 4
            + [pltpu.VMEM((_B, _H), jnp.float32)] * 2
            + [pltpu.VMEM((_B, _D), jnp.float32)] * 2
            + [pltpu.VMEM_SHARED((_N, _D), jnp.float32)]
            + [pltpu.SemaphoreType.DMA] * 12
        ),
    )
    return k(src, dst, stbl, dtbl, q)


def kernel(x, edge_index, edge_attr, fcn_W, fcn_b, fce_W, fce_b, bn_g, bn_b,
           fc_W0, fc_b0, attn_W0, attn_b0,
           fc_W1, fc_b1, attn_W1, attn_b1,
           fc_W2, fc_b2, attn_W2, attn_b2):
    src = edge_index[0].astype(jnp.int32)
    dst = edge_index[1].astype(jnp.int32)

    fcW = jnp.stack([fc_W0, fc_W1, fc_W2])
    fcb = jnp.stack([fc_b0, fc_b1, fc_b2])[:, None, :]
    aW = jnp.stack([attn_W0, attn_W1, attn_W2])
    Ai = aW[:, :, :_D]
    Aj = aW[:, :, _D:2 * _D]
    Ae = aW[:, :, 2 * _D:]
    ab = jnp.stack([attn_b0, attn_b1, attn_b2])[:, None, :]

    Wh, bh, Wi, bi, Wj, bj, M, cq = _fold(
        x, fcn_W, fcn_b[None, :], bn_g[None, :], bn_b[None, :],
        fce_W, fce_b[None, :], fcW, fcb, Ai, Aj, Ae, ab)
    q0, q1, q2 = _qkern(edge_attr, M, cq)
    qs = [q0.reshape(2 * _E, _H), q1.reshape(2 * _E, _H),
          q2.reshape(2 * _E, _H)]

    acc = None
    for l in range(3):
        h_in = x if l == 0 else acc.reshape(2, _N, _D)
        stbl, dtbl = _tables(h_in, Wh[l], bh[l], Wi[l], bi[l], Wj[l], bj[l],
                             first=(l == 0))
        acc = _sc_pass(src, dst, stbl.reshape(2 * _N, _D), dtbl, qs[l])

    out = _final(acc.reshape(2, _N, _D))
    return out.reshape(1, _N, _D)


# speedup vs baseline: 7.2564x; 1.0002x over previous
import jax
import jax.numpy as jnp
from jax import lax
from jax.experimental import pallas as pl
from jax.experimental.pallas import tpu as pltpu
from jax.experimental.pallas import tpu_sc as plsc

_N = 10000
_E = 320000
_D = 128
_DE = 16
_H = _D // 2

_NC = 2
_NS = 16
_B = 32
_EPT = _E // _NS
_ZR = 632
_LASTR = _N - (_NS - 1) * _ZR

_BN_BLK = 1000
_BE_BLK = 4000


def _fold_body(x_ref, fcnW_ref, fcnb_ref, bng_ref, bnb_ref,
               fceW_ref, fceb_ref, fcW_ref, fcb_ref,
               Ai_ref, Aj_ref, Ae_ref, ab_ref,
               Wh_ref, bh_ref, Wi_ref, bi_ref, Wj_ref, bj_ref,
               M_ref, cq_ref):
    n = jnp.float32(_N)
    xb = x_ref[...]
    S = lax.dot_general(xb, xb, (((0,), (0,)), ((), ())),
                        preferred_element_type=jnp.float32)
    mx = jnp.sum(xb, axis=0, keepdims=True) / n
    C = S / n - mx.T @ mx
    fcnW = fcnW_ref[...]
    mh = mx @ fcnW.T + fcnb_ref[...]
    t1 = fcnW @ C
    var = jnp.sum(t1 * fcnW, axis=1, keepdims=True).T
    scale = bng_ref[...] / jnp.sqrt(var + 1e-5)
    shift = bnb_ref[...] - mh * scale
    W2 = fcnW.T * scale
    b2 = fcnb_ref[...] * scale + shift
    fceW = fceW_ref[...]
    fceb = fceb_ref[...]
    for l in range(3):
        fcW = fcW_ref[l]
        fcb = fcb_ref[l]
        if l == 0:
            Wh = W2 @ fcW.T
            bh = b2 @ fcW.T + fcb
        else:
            Wh = fcW.T
            bh = fcb
        Ai = Ai_ref[l]
        Aj = Aj_ref[l]
        Ae = Ae_ref[l]
        Wh_ref[l] = Wh
        bh_ref[l] = bh
        Wi_ref[l] = Wh @ Ai.T
        bi_ref[l] = bh @ Ai.T
        Wj_ref[l] = Wh @ Aj.T
        bj_ref[l] = bh @ Aj.T
        M_ref[l] = (Ae @ fceW).T
        cq_ref[l] = fceb @ Ae.T + ab_ref[l]


def _fold(x, fcnW, fcnb, bng, bnb, fceW, fceb, fcW, fcb, Ai, Aj, Ae, ab):
    full = lambda shp: pl.BlockSpec(shp, lambda: tuple(0 for _ in shp))
    ins = [x, fcnW, fcnb, bng, bnb, fceW, fceb, fcW, fcb, Ai, Aj, Ae, ab]
    return pl.pallas_call(
        _fold_body,
        in_specs=[full(a.shape) for a in ins],
        out_specs=[full((3, _D, _D)), full((3, 1, _D)),
                   full((3, _D, _D)), full((3, 1, _D)),
                   full((3, _D, _D)), full((3, 1, _D)),
                   full((3, _DE, _D)), full((3, 1, _D))],
        out_shape=[jax.ShapeDtypeStruct((3, _D, _D), jnp.float32),
                   jax.ShapeDtypeStruct((3, 1, _D), jnp.float32),
                   jax.ShapeDtypeStruct((3, _D, _D), jnp.float32),
                   jax.ShapeDtypeStruct((3, 1, _D), jnp.float32),
                   jax.ShapeDtypeStruct((3, _D, _D), jnp.float32),
                   jax.ShapeDtypeStruct((3, 1, _D), jnp.float32),
                   jax.ShapeDtypeStruct((3, _DE, _D), jnp.float32),
                   jax.ShapeDtypeStruct((3, 1, _D), jnp.float32)],
        interpret=False,
    )(*ins)


def _q_body(ea_ref, M_ref, cq_ref, q0_ref, q1_ref, q2_ref):
    ea = ea_ref[...]
    for l, qr in enumerate((q0_ref, q1_ref, q2_ref)):
        q = ea @ M_ref[l] + cq_ref[l]
        qr[0] = q[:, :_H]
        qr[1] = q[:, _H:]


def _qkern(ea, M, cq):
    qspec = pl.BlockSpec((2, _BE_BLK, _H), lambda i: (0, i, 0))
    qshape = jax.ShapeDtypeStruct((2, _E, _H), jnp.float32)
    return pl.pallas_call(
        _q_body,
        grid=(_E // _BE_BLK,),
        in_specs=[pl.BlockSpec((_BE_BLK, _DE), lambda i: (i, 0)),
                  pl.BlockSpec((3, _DE, _D), lambda i: (0, 0, 0)),
                  pl.BlockSpec((3, 1, _D), lambda i: (0, 0, 0))],
        out_specs=[qspec, qspec, qspec],
        out_shape=[qshape, qshape, qshape],
        interpret=False,
    )(ea, M, cq)


def _tables0_body(x_ref, Wh_ref, bh_ref, Wi_ref, bi_ref, Wj_ref, bj_ref,
                  stbl_ref, dtbl_ref):
    x = x_ref[...]
    h = x @ Wh_ref[...] + bh_ref[...]
    piv = x @ Wi_ref[...] + bi_ref[...]
    pjv = x @ Wj_ref[...] + bj_ref[...]
    stbl_ref[0] = jnp.concatenate([h[:, :_H], pjv[:, :_H]], axis=1)
    stbl_ref[1] = jnp.concatenate([h[:, _H:], pjv[:, _H:]], axis=1)
    dtbl_ref[...] = piv


def _tablesL_body(acc_ref, Wh_ref, bh_ref, Wi_ref, bi_ref, Wj_ref, bj_ref,
                  stbl_ref, dtbl_ref):
    a0 = acc_ref[0]
    a1 = acc_ref[1]
    x = jnp.concatenate([a0[:, :_H] / (a0[:, _H:] + 1e-16),
                         a1[:, :_H] / (a1[:, _H:] + 1e-16)], axis=1)
    h = x @ Wh_ref[...] + bh_ref[...]
    piv = x @ Wi_ref[...] + bi_ref[...]
    pjv = x @ Wj_ref[...] + bj_ref[...]
    stbl_ref[0] = jnp.concatenate([h[:, :_H], pjv[:, :_H]], axis=1)
    stbl_ref[1] = jnp.concatenate([h[:, _H:], pjv[:, _H:]], axis=1)
    dtbl_ref[...] = piv


def _tables(h_or_acc, Wh, bh, Wi, bi, Wj, bj, first):
    body = _tables0_body if first else _tablesL_body
    in0 = (pl.BlockSpec((_BN_BLK, _D), lambda i: (i, 0)) if first
           else pl.BlockSpec((2, _BN_BLK, _D), lambda i: (0, i, 0)))
    wspec = pl.BlockSpec((_D, _D), lambda i: (0, 0))
    bspec = pl.BlockSpec((1, _D), lambda i: (0, 0))
    return pl.pallas_call(
        body,
        grid=(_N // _BN_BLK,),
        in_specs=[in0, wspec, bspec, wspec, bspec, wspec, bspec],
        out_specs=[pl.BlockSpec((2, _BN_BLK, _D), lambda i: (0, i, 0)),
                   pl.BlockSpec((_BN_BLK, _D), lambda i: (i, 0))],
        out_shape=[jax.ShapeDtypeStruct((2, _N, _D), jnp.float32),
                   jax.ShapeDtypeStruct((_N, _D), jnp.float32)],
        interpret=False,
    )(h_or_acc, Wh, bh, Wi, bi, Wj, bj)


def _final_body(acc_ref, out_ref):
    a0 = acc_ref[0]
    a1 = acc_ref[1]
    out_ref[...] = jnp.concatenate([a0[:, :_H] / (a0[:, _H:] + 1e-16),
                                    a1[:, :_H] / (a1[:, _H:] + 1e-16)], axis=1)


def _final(acc):
    return pl.pallas_call(
        _final_body,
        grid=(_N // _BN_BLK,),
        in_specs=[pl.BlockSpec((2, _BN_BLK, _D), lambda i: (0, i, 0))],
        out_specs=pl.BlockSpec((_BN_BLK, _D), lambda i: (i, 0)),
        out_shape=jax.ShapeDtypeStruct((_N, _D), jnp.float32),
        interpret=False,
    )(acc)


def _sc_pass_body(src_ref, dst_ref, stbl_ref, dtbl_ref, q_ref, out_ref,
                  si0, si1, di0, di1, db0, db1, r0, r1, p0, p1, qb0, qb1,
                  o0, o1, acc,
                  sa0, sa1, sd0, sd1, sr0, sr1, sp0, sp1, sq0, sq1, ss0, ss1):
    c = lax.axis_index("c")
    s = lax.axis_index("s")
    coff = c * _N
    choff = c * _H
    nch = _EPT // _B
    ebase = s * _EPT
    qbase = c * _E + ebase

    si_b = (si0, si1)
    di_b = (di0, di1)
    db_b = (db0, db1)
    rows_b = (r0, r1)
    pd_b = (p0, p1)
    qv_b = (qb0, qb1)
    ov_b = (o0, o1)
    sem_si = (sa0, sa1)
    sem_di = (sd0, sd1)
    sem_r = (sr0, sr1)
    sem_p = (sp0, sp1)
    sem_q = (sq0, sq1)
    sem_s = (ss0, ss1)

    def zf(i, _):
        for k in range(_D // 16):
            o0[i, pl.ds(16 * k, 16)] = jnp.zeros((16,), jnp.float32)
        return 0
    lax.fori_loop(0, _B, zf, 0)

    nz8 = jnp.where(s == _NS - 1, _LASTR // 8, _ZR // 8)

    def zc(i, _):
        pltpu.async_copy(o0.at[pl.ds(0, 8)],
                         acc.at[pl.ds(s * _ZR + 8 * i, 8)], sa0)
        return 0
    lax.fori_loop(0, nz8, zc, 0)

    def zw(i, _):
        pltpu.make_async_copy(o0.at[pl.ds(0, 8)],
                              acc.at[pl.ds(s * _ZR, 8)], sa0).wait()
        return 0
    lax.fori_loop(0, nz8, zw, 0)
    plsc.subcore_barrier()

    def issue_idx(j, t):
        off = ebase + j * _B
        pltpu.async_copy(src_ref.at[pl.ds(off, _B)], si_b[t], sem_si[t])
        pltpu.async_copy(dst_ref.at[pl.ds(off, _B)], di_b[t], sem_di[t])

    def wait_idx_and_adjust(t):
        pltpu.make_async_copy(src_ref.at[pl.ds(0, _B)], si_b[t],
                              sem_si[t]).wait()
        pltpu.make_async_copy(dst_ref.at[pl.ds(0, _B)], di_b[t],
                              sem_di[t]).wait()
        for k in range(_B // 16):
            sl = pl.ds(16 * k, 16)
            si_b[t][sl] = si_b[t][sl] + coff

    def issue_gathers(j, t):
        pltpu.async_copy(stbl_ref.at[si_b[t]], rows_b[t], sem_r[t])
        pltpu.async_copy(dtbl_ref.at[di_b[t]], pd_b[t], sem_p[t])
        pltpu.async_copy(q_ref.at[pl.ds(qbase + j * _B, _B)], qv_b[t],
                         sem_q[t])

    def wait_gathers(t):
        pltpu.make_async_copy(stbl_ref.at[si_b[t]], rows_b[t], sem_r[t]).wait()
        pltpu.make_async_copy(dtbl_ref.at[di_b[t]], pd_b[t], sem_p[t]).wait()
        pltpu.make_async_copy(q_ref.at[pl.ds(qbase, _B)], qv_b[t],
                              sem_q[t]).wait()

    def compute(t):
        rr = rows_b[t]
        pp = pd_b[t]
        qq = qv_b[t]
        oo = ov_b[t]

        @plsc.parallel_loop(0, _B, 1, unroll=2)
        def _(i2):
            for k in range(_H // 16):
                lo = pl.ds(16 * k, 16)
                hi = pl.ds(_H + 16 * k, 16)
                a = pp[i2, pl.ds(choff + 16 * k, 16)] + rr[i2, hi] + qq[i2, lo]
                a = jnp.maximum(a, a * 0.2)
                ex = jnp.exp(a)
                oo[i2, lo] = rr[i2, lo] * ex
                oo[i2, hi] = ex

    def wait_scatter(t):
        pltpu.make_async_copy(ov_b[t], acc.at[db_b[t]], sem_s[t]).wait()

    def copy_db(t):
        for k in range(_B // 16):
            sl = pl.ds(16 * k, 16)
            db_b[t][sl] = di_b[t][sl]

    issue_idx(0, 0)
    wait_idx_and_adjust(0)
    issue_gathers(0, 0)
    issue_idx(1, 1)

    def body(i, _):
        for b in (0, 1):
            j = 2 * i + b
            t = b
            nt = 1 - b
            wait_idx_and_adjust(nt)
            issue_gathers(j + 1, nt)

            @pl.when(i > 0)
            def _():
                wait_scatter(t)
            wait_gathers(t)
            copy_db(t)
            if b == 0:
                issue_idx(j + 2, t)
            else:
                @pl.when(i < (nch - 1) // 2 - 1)
                def _():
                    issue_idx(j + 2, t)
            compute(t)
            pltpu.async_copy(ov_b[t], acc.at[db_b[t]], sem_s[t], add=True)
        return 0
    lax.fori_loop(0, (nch - 1) // 2, body, 0)

    wait_scatter(0)
    wait_gathers(0)
    copy_db(0)
    compute(0)
    pltpu.async_copy(ov_b[0], acc.at[db_b[0]], sem_s[0], add=True)
    wait_scatter(1)
    wait_scatter(0)
    plsc.subcore_barrier()

    @pl.when(s < _NS - 1)
    def _():
        pltpu.sync_copy(acc.at[pl.ds(s * _ZR, _ZR)],
                        out_ref.at[pl.ds(coff + s * _ZR, _ZR)])

    @pl.when(s == _NS - 1)
    def _():
        pltpu.sync_copy(acc.at[pl.ds((_NS - 1) * _ZR, _LASTR)],
                        out_ref.at[pl.ds(coff + (_NS - 1) * _ZR, _LASTR)])


def _sc_pass(src, dst, stbl, dtbl, q):
    mesh = plsc.VectorSubcoreMesh(core_axis_name="c", subcore_axis_name="s",
                                  num_cores=_NC, num_subcores=_NS)
    k = pl.kernel(
        _sc_pass_body,
        out_type=jax.ShapeDtypeStruct((2 * _N, _D), jnp.float32),
        mesh=mesh,
        scratch_types=(
            [pltpu.VMEM((_B,), jnp.int32)] * 6
            + [pltpu.VMEM((_B, _D), jnp.float32)] * 4
            + [pltpu.VMEM((_B, _H), jnp.float32)] * 2
            + [pltpu.VMEM((_B, _D), jnp.float32)] * 2
            + [pltpu.VMEM_SHARED((_N, _D), jnp.float32)]
            + [pltpu.SemaphoreType.DMA] * 12
        ),
    )
    return k(src, dst, stbl, dtbl, q)


def kernel(x, edge_index, edge_attr, fcn_W, fcn_b, fce_W, fce_b, bn_g, bn_b,
           fc_W0, fc_b0, attn_W0, attn_b0,
           fc_W1, fc_b1, attn_W1, attn_b1,
           fc_W2, fc_b2, attn_W2, attn_b2):
    src = edge_index[0].astype(jnp.int32)
    dst = edge_index[1].astype(jnp.int32)

    fcW = jnp.stack([fc_W0, fc_W1, fc_W2])
    fcb = jnp.stack([fc_b0, fc_b1, fc_b2])[:, None, :]
    aW = jnp.stack([attn_W0, attn_W1, attn_W2])
    Ai = aW[:, :, :_D]
    Aj = aW[:, :, _D:2 * _D]
    Ae = aW[:, :, 2 * _D:]
    ab = jnp.stack([attn_b0, attn_b1, attn_b2])[:, None, :]

    Wh, bh, Wi, bi, Wj, bj, M, cq = _fold(
        x, fcn_W, fcn_b[None, :], bn_g[None, :], bn_b[None, :],
        fce_W, fce_b[None, :], fcW, fcb, Ai, Aj, Ae, ab)
    q0, q1, q2 = _qkern(edge_attr, M, cq)
    qs = [q0.reshape(2 * _E, _H), q1.reshape(2 * _E, _H),
          q2.reshape(2 * _E, _H)]

    acc = None
    for l in range(3):
        h_in = x if l == 0 else acc.reshape(2, _N, _D)
        stbl, dtbl = _tables(h_in, Wh[l], bh[l], Wi[l], bi[l], Wj[l], bj[l],
                             first=(l == 0))
        acc = _sc_pass(src, dst, stbl.reshape(2 * _N, _D), dtbl, qs[l])

    out = _final(acc.reshape(2, _N, _D))
    return out.reshape(1, _N, _D)
